# baseline (speedup 1.0000x reference)
import functools

import jax
import jax.numpy as jnp
from jax import lax
from jax.experimental import pallas as pl
from jax.experimental.pallas import tpu as pltpu
from jax.experimental.pallas import tpu_sc as plsc

_BATCH = 16384
_EMB = 32
_NW = 32
_BPW = _BATCH // _NW
_BB = 2048


def _gather_sc(user_emb, item_emb, users, items):
    mesh = plsc.VectorSubcoreMesh(core_axis_name="c", subcore_axis_name="s")

    @functools.partial(
        pl.kernel,
        mesh=mesh,
        compiler_params=pltpu.CompilerParams(use_tc_tiling_on_sc=False),
        out_type=[
            jax.ShapeDtypeStruct((_BATCH, _EMB), jnp.float32),
            jax.ShapeDtypeStruct((_BATCH, _EMB), jnp.float32),
        ],
        scratch_types=[
            pltpu.VMEM((_BPW,), jnp.int32),
            pltpu.VMEM((_BPW, _EMB), jnp.float32),
            pltpu.VMEM((_BPW,), jnp.int32),
            pltpu.VMEM((_BPW, _EMB), jnp.float32),
            pltpu.SemaphoreType.DMA,
            pltpu.SemaphoreType.DMA,
        ],
    )
    def k(uemb_hbm, iemb_hbm, users_hbm, items_hbm, ue_out, ie_out,
          uidx_v, urows_v, iidx_v, irows_v, su, si):
        wid = lax.axis_index("s") * 2 + lax.axis_index("c")
        base = wid * _BPW
        pltpu.sync_copy(users_hbm.at[pl.ds(base, _BPW)], uidx_v)
        pltpu.sync_copy(items_hbm.at[pl.ds(base, _BPW)], iidx_v)
        cu = pltpu.async_copy(uemb_hbm.at[uidx_v], urows_v, su)
        ci = pltpu.async_copy(iemb_hbm.at[iidx_v], irows_v, si)
        cu.wait()
        ci.wait()
        pltpu.sync_copy(urows_v, ue_out.at[pl.ds(base, _BPW)])
        pltpu.sync_copy(irows_v, ie_out.at[pl.ds(base, _BPW)])

    return k(user_emb, item_emb, users, items)


def _mlp_body(ue_ref, ie_ref, w1u, w1i, b1r, w2, b2r, w3r, b3r, out_ref):
    x = jnp.dot(ue_ref[...], w1u[...], preferred_element_type=jnp.float32)
    x = x + jnp.dot(ie_ref[...], w1i[...], preferred_element_type=jnp.float32)
    h1 = jnp.maximum(x + b1r[...], 0.0)
    h2 = jnp.maximum(
        jnp.dot(h1, w2[...], preferred_element_type=jnp.float32) + b2r[...], 0.0
    )
    out_ref[...] = jnp.sum(h2 * w3r[...], axis=1) + b3r[0]


def _mlp_tc(ue, ie, W1, b1, W2, b2, W3, b3):
    W1u = W1[:_EMB]
    W1i = W1[_EMB:]
    w3row = W3[:, 0]
    grid = (_BATCH // _BB,)
    return pl.pallas_call(
        _mlp_body,
        grid=grid,
        in_specs=[
            pl.BlockSpec((_BB, _EMB), lambda i: (i, 0)),
            pl.BlockSpec((_BB, _EMB), lambda i: (i, 0)),
            pl.BlockSpec((_EMB, 128), lambda i: (0, 0)),
            pl.BlockSpec((_EMB, 128), lambda i: (0, 0)),
            pl.BlockSpec((128,), lambda i: (0,)),
            pl.BlockSpec((128, 64), lambda i: (0, 0)),
            pl.BlockSpec((64,), lambda i: (0,)),
            pl.BlockSpec((64,), lambda i: (0,)),
            pl.BlockSpec((1,), lambda i: (0,)),
        ],
        out_specs=pl.BlockSpec((_BB,), lambda i: (i,)),
        out_shape=jax.ShapeDtypeStruct((_BATCH,), jnp.float32),
    )(ue, ie, W1u, W1i, b1, W2, b2, w3row, b3)


def kernel(users, items, user_emb, item_emb, W1, b1, W2, b2, W3, b3):
    users = users.astype(jnp.int32)
    items = items.astype(jnp.int32)
    ue, ie = _gather_sc(user_emb, item_emb, users, items)
    return _mlp_tc(ue, ie, W1, b1, W2, b2, W3, b3)


# speedup vs baseline: 1.6963x; 1.6963x over previous
import functools

import jax
import jax.numpy as jnp
from jax import lax
from jax.experimental import pallas as pl
from jax.experimental.pallas import tpu as pltpu
from jax.experimental.pallas import tpu_sc as plsc

_BATCH = 16384
_EMB = 32
_NW = 32
_BPW = _BATCH // _NW
_RND = 2
_HB = _BPW // _RND
_BB = 1024


def _t_body(n, in_ref, out_ref):
    x = in_ref[...]
    L = x.shape[1]
    lane = jax.lax.broadcasted_iota(jnp.int32, x.shape, 1) + pl.program_id(0) * L
    x = jnp.where(lane < n, x, 0.0)
    q = L // 4
    eye = jnp.eye(_EMB, dtype=jnp.float32)
    parts = [
        lax.dot_general(x[:, j * q:(j + 1) * q], eye, (((0,), (0,)), ((), ())),
                        preferred_element_type=jnp.float32)
        for j in range(4)
    ]
    out_ref[...] = jnp.concatenate(parts, axis=1)


def _transpose_tc(embT, L=8192):
    n = embT.shape[1]
    nblk = (n + L - 1) // L
    grid = (nblk,)
    return pl.pallas_call(
        functools.partial(_t_body, n),
        grid=grid,
        in_specs=[pl.BlockSpec((_EMB, L), lambda m: (0, m))],
        out_specs=pl.BlockSpec((L // 4, 128), lambda m: (m, 0)),
        out_shape=jax.ShapeDtypeStruct((nblk * L // 4, 128), jnp.float32),
    )(embT)


def _gather_sc(u4, i4, users, items):
    mesh = plsc.VectorSubcoreMesh(core_axis_name="c", subcore_axis_name="s")

    @functools.partial(
        pl.kernel,
        mesh=mesh,
        out_type=[
            jax.ShapeDtypeStruct((_BATCH, 128), jnp.float32),
            jax.ShapeDtypeStruct((_BATCH, 128), jnp.float32),
        ],
        scratch_types=[
            pltpu.VMEM((_BPW,), jnp.int32),
            pltpu.VMEM((_BPW,), jnp.int32),
            pltpu.VMEM((_HB, 128), jnp.float32),
            pltpu.VMEM((_HB, 128), jnp.float32),
            pltpu.SemaphoreType.DMA,
            pltpu.SemaphoreType.DMA,
        ],
    )
    def k(u4_hbm, i4_hbm, users_hbm, items_hbm, ru_out, ri_out,
          utid, itid, ubuf, ibuf, su, si):
        wid = lax.axis_index("s") * 2 + lax.axis_index("c")
        base = wid * _BPW
        pltpu.sync_copy(users_hbm.at[pl.ds(base, _BPW)], utid)
        pltpu.sync_copy(items_hbm.at[pl.ds(base, _BPW)], itid)
        for g in range(_BPW // 16):
            tu = utid[pl.ds(g * 16, 16)]
            utid[pl.ds(g * 16, 16)] = ((tu >> 13) << 11) | (tu & 2047)
            ti = itid[pl.ds(g * 16, 16)]
            itid[pl.ds(g * 16, 16)] = ((ti >> 13) << 11) | (ti & 2047)

        def round_body(h, _):
            h0 = h * _HB
            cu = pltpu.async_copy(
                u4_hbm.at[utid.at[pl.ds(h0, _HB)]], ubuf, su)
            ci = pltpu.async_copy(
                i4_hbm.at[itid.at[pl.ds(h0, _HB)]], ibuf, si)
            cu.wait()
            ci.wait()
            pltpu.sync_copy(ubuf, ru_out.at[pl.ds(base + h0, _HB)])
            pltpu.sync_copy(ibuf, ri_out.at[pl.ds(base + h0, _HB)])
            return _

        lax.fori_loop(0, _RND, round_body, 0)

    return k(u4, i4, users, items)


def _mlp_body(ru_ref, ri_ref, uk_ref, ik_ref, w1e, b1r, w2, b2r, w3r, b3r,
              out_ref):
    ru = ru_ref[...]
    ri = ri_ref[...]
    uk = (uk_ref[...] >> 11) & 3
    ik = (ik_ref[...] >> 11) & 3
    parts = []
    for k in range(4):
        m = (uk == k).astype(jnp.float32)
        parts.append(ru * m[:, None])
    for k in range(4):
        m = (ik == k).astype(jnp.float32)
        parts.append(ri * m[:, None])
    xcat = jnp.concatenate(parts, axis=1)
    h1 = jnp.maximum(
        jnp.dot(xcat, w1e[...], preferred_element_type=jnp.float32)
        + b1r[...], 0.0)
    h2 = jnp.maximum(
        jnp.dot(h1, w2[...], preferred_element_type=jnp.float32) + b2r[...],
        0.0)
    out_ref[...] = jnp.sum(h2 * w3r[...], axis=1) + b3r[0]


def _mlp_tc(ru, ri, users, items, W1, b1, W2, b2, W3, b3):
    w1e = jnp.zeros((1024, 128), jnp.float32)
    for k in range(4):
        w1e = w1e.at[128 * k + 32 * k: 128 * k + 32 * k + 32].set(W1[:_EMB])
        w1e = w1e.at[512 + 128 * k + 32 * k: 512 + 128 * k + 32 * k + 32].set(
            W1[_EMB:])
    w3row = W3[:, 0]
    grid = (_BATCH // _BB,)
    return pl.pallas_call(
        _mlp_body,
        grid=grid,
        in_specs=[
            pl.BlockSpec((_BB, 128), lambda i: (i, 0)),
            pl.BlockSpec((_BB, 128), lambda i: (i, 0)),
            pl.BlockSpec((_BB,), lambda i: (i,)),
            pl.BlockSpec((_BB,), lambda i: (i,)),
            pl.BlockSpec((1024, 128), lambda i: (0, 0)),
            pl.BlockSpec((128,), lambda i: (0,)),
            pl.BlockSpec((128, 64), lambda i: (0, 0)),
            pl.BlockSpec((64,), lambda i: (0,)),
            pl.BlockSpec((64,), lambda i: (0,)),
            pl.BlockSpec((1,), lambda i: (0,)),
        ],
        out_specs=pl.BlockSpec((_BB,), lambda i: (i,)),
        out_shape=jax.ShapeDtypeStruct((_BATCH,), jnp.float32),
    )(ru, ri, users, items, w1e, b1, W2, b2, w3row, b3)


def kernel(users, items, user_emb, item_emb, W1, b1, W2, b2, W3, b3):
    users = users.astype(jnp.int32)
    items = items.astype(jnp.int32)
    u4 = _transpose_tc(user_emb.T)
    i4 = _transpose_tc(item_emb.T)
    ru, ri = _gather_sc(u4, i4, users, items)
    return _mlp_tc(ru, ri, users, items, W1, b1, W2, b2, W3, b3)


# speedup vs baseline: 2.6405x; 1.5566x over previous
import functools

import jax
import jax.numpy as jnp
from jax import lax
from jax.experimental import pallas as pl
from jax.experimental.pallas import tpu as pltpu
from jax.experimental.pallas import tpu_sc as plsc

_BATCH = 16384
_EMB = 32
_NW = 32
_BPW = _BATCH // _NW
_RND = 2
_HB = _BPW // _RND
_BB = 1024


def _t_body(in_ref, out_ref):
    x = in_ref[...]
    L = x.shape[1]
    q = L // 4
    xs = jnp.concatenate([x[:, j * q:(j + 1) * q] for j in range(4)], axis=0)
    out_ref[...] = jnp.transpose(xs, (1, 0))


def _transpose_tc(embT, L=8192):
    n = embT.shape[1]
    nblk = (n + L - 1) // L
    grid = (nblk,)
    return pl.pallas_call(
        _t_body,
        grid=grid,
        in_specs=[pl.BlockSpec((_EMB, L), lambda m: (0, m))],
        out_specs=pl.BlockSpec((L // 4, 128), lambda m: (m, 0)),
        out_shape=jax.ShapeDtypeStruct((nblk * L // 4, 128), jnp.float32),
    )(embT)


def _gather_sc(u4, i4, users, items):
    mesh = plsc.VectorSubcoreMesh(core_axis_name="c", subcore_axis_name="s")

    @functools.partial(
        pl.kernel,
        mesh=mesh,
        out_type=[
            jax.ShapeDtypeStruct((_BATCH, 128), jnp.float32),
            jax.ShapeDtypeStruct((_BATCH, 128), jnp.float32),
        ],
        scratch_types=[
            pltpu.VMEM((_BPW,), jnp.int32),
            pltpu.VMEM((_BPW,), jnp.int32),
            pltpu.VMEM((_HB, 128), jnp.float32),
            pltpu.VMEM((_HB, 128), jnp.float32),
            pltpu.SemaphoreType.DMA,
            pltpu.SemaphoreType.DMA,
        ],
    )
    def k(u4_hbm, i4_hbm, users_hbm, items_hbm, ru_out, ri_out,
          utid, itid, ubuf, ibuf, su, si):
        wid = lax.axis_index("s") * 2 + lax.axis_index("c")
        base = wid * _BPW
        pltpu.sync_copy(users_hbm.at[pl.ds(base, _BPW)], utid)
        pltpu.sync_copy(items_hbm.at[pl.ds(base, _BPW)], itid)
        for g in range(_BPW // 16):
            tu = utid[pl.ds(g * 16, 16)]
            utid[pl.ds(g * 16, 16)] = ((tu >> 13) << 11) | (tu & 2047)
            ti = itid[pl.ds(g * 16, 16)]
            itid[pl.ds(g * 16, 16)] = ((ti >> 13) << 11) | (ti & 2047)

        def round_body(h, _):
            h0 = h * _HB
            cu = pltpu.async_copy(
                u4_hbm.at[utid.at[pl.ds(h0, _HB)]], ubuf, su)
            ci = pltpu.async_copy(
                i4_hbm.at[itid.at[pl.ds(h0, _HB)]], ibuf, si)
            cu.wait()
            ci.wait()
            pltpu.sync_copy(ubuf, ru_out.at[pl.ds(base + h0, _HB)])
            pltpu.sync_copy(ibuf, ri_out.at[pl.ds(base + h0, _HB)])
            return _

        lax.fori_loop(0, _RND, round_body, 0)

    return k(u4, i4, users, items)


def _mlp_body(ru_ref, ri_ref, uk_ref, ik_ref, w1e, b1r, w2, b2r, w3r, b3r,
              out_ref):
    ru = ru_ref[...]
    ri = ri_ref[...]
    uk = (uk_ref[...] >> 11) & 3
    ik = (ik_ref[...] >> 11) & 3
    lane = jax.lax.broadcasted_iota(jnp.int32, (1, 128), 1)
    parts = []
    for k in range(4):
        keep = (uk[:, None] == k) & (lane >> 5 == k)
        parts.append(jnp.where(keep, ru, 0.0))
    for k in range(4):
        keep = (ik[:, None] == k) & (lane >> 5 == k)
        parts.append(jnp.where(keep, ri, 0.0))
    xcat = jnp.concatenate(parts, axis=1)
    h1 = jnp.maximum(
        jnp.dot(xcat, w1e[...], preferred_element_type=jnp.float32)
        + b1r[...], 0.0)
    h2 = jnp.maximum(
        jnp.dot(h1, w2[...], preferred_element_type=jnp.float32) + b2r[...],
        0.0)
    out_ref[...] = jnp.sum(h2 * w3r[...], axis=1) + b3r[0]


def _mlp_tc(ru, ri, users, items, W1, b1, W2, b2, W3, b3):
    w1e = jnp.zeros((1024, 128), jnp.float32)
    for k in range(4):
        w1e = w1e.at[128 * k + 32 * k: 128 * k + 32 * k + 32].set(W1[:_EMB])
        w1e = w1e.at[512 + 128 * k + 32 * k: 512 + 128 * k + 32 * k + 32].set(
            W1[_EMB:])
    w3row = W3[:, 0]
    grid = (_BATCH // _BB,)
    return pl.pallas_call(
        _mlp_body,
        grid=grid,
        in_specs=[
            pl.BlockSpec((_BB, 128), lambda i: (i, 0)),
            pl.BlockSpec((_BB, 128), lambda i: (i, 0)),
            pl.BlockSpec((_BB,), lambda i: (i,)),
            pl.BlockSpec((_BB,), lambda i: (i,)),
            pl.BlockSpec((1024, 128), lambda i: (0, 0)),
            pl.BlockSpec((128,), lambda i: (0,)),
            pl.BlockSpec((128, 64), lambda i: (0, 0)),
            pl.BlockSpec((64,), lambda i: (0,)),
            pl.BlockSpec((64,), lambda i: (0,)),
            pl.BlockSpec((1,), lambda i: (0,)),
        ],
        out_specs=pl.BlockSpec((_BB,), lambda i: (i,)),
        out_shape=jax.ShapeDtypeStruct((_BATCH,), jnp.float32),
    )(ru, ri, users, items, w1e, b1, W2, b2, w3row, b3)


def kernel(users, items, user_emb, item_emb, W1, b1, W2, b2, W3, b3):
    users = users.astype(jnp.int32)
    items = items.astype(jnp.int32)
    u4 = _transpose_tc(user_emb.T)
    i4 = _transpose_tc(item_emb.T)
    ru, ri = _gather_sc(u4, i4, users, items)
    return _mlp_tc(ru, ri, users, items, W1, b1, W2, b2, W3, b3)


# speedup vs baseline: 3.5339x; 1.3384x over previous
import functools

import jax
import jax.numpy as jnp
from jax import lax
from jax.experimental import pallas as pl
from jax.experimental.pallas import tpu as pltpu
from jax.experimental.pallas import tpu_sc as plsc

_BATCH = 16384
_EMB = 32
_L = 32768
_LBITS = 15
_QBITS = _LBITS - 2
_NW = 32
_BPW = _BATCH // _NW
_RND = 2
_HB = _BPW // _RND
_BB = 1024


def _t_body(in_ref, out_ref):
    x = in_ref[...]
    L = x.shape[1]
    q = L // 4
    xs = jnp.concatenate([x[:, j * q:(j + 1) * q] for j in range(4)], axis=0)
    out_ref[...] = jnp.transpose(xs, (1, 0))


def _transpose_tc(embT, L=_L):
    n = embT.shape[1]
    nblk = (n + L - 1) // L
    grid = (nblk,)
    return pl.pallas_call(
        _t_body,
        grid=grid,
        in_specs=[pl.BlockSpec((_EMB, L), lambda m: (0, m))],
        out_specs=pl.BlockSpec((L // 4, 128), lambda m: (m, 0)),
        out_shape=jax.ShapeDtypeStruct((nblk * L // 4, 128), jnp.float32),
    )(embT)


def _gather_sc(u4, i4, users, items):
    mesh = plsc.VectorSubcoreMesh(core_axis_name="c", subcore_axis_name="s")

    @functools.partial(
        pl.kernel,
        mesh=mesh,
        out_type=[
            jax.ShapeDtypeStruct((_BATCH, 128), jnp.float32),
            jax.ShapeDtypeStruct((_BATCH, 128), jnp.float32),
        ],
        scratch_types=[
            pltpu.VMEM((_BPW,), jnp.int32),
            pltpu.VMEM((_BPW,), jnp.int32),
            pltpu.VMEM((_HB, 128), jnp.float32),
            pltpu.VMEM((_HB, 128), jnp.float32),
            pltpu.SemaphoreType.DMA,
            pltpu.SemaphoreType.DMA,
        ],
    )
    def k(u4_hbm, i4_hbm, users_hbm, items_hbm, ru_out, ri_out,
          utid, itid, ubuf, ibuf, su, si):
        wid = lax.axis_index("s") * 2 + lax.axis_index("c")
        base = wid * _BPW
        pltpu.sync_copy(users_hbm.at[pl.ds(base, _BPW)], utid)
        pltpu.sync_copy(items_hbm.at[pl.ds(base, _BPW)], itid)
        for g in range(_BPW // 16):
            tu = utid[pl.ds(g * 16, 16)]
            utid[pl.ds(g * 16, 16)] = ((tu >> _LBITS) << _QBITS) | (tu & (_L // 4 - 1))
            ti = itid[pl.ds(g * 16, 16)]
            itid[pl.ds(g * 16, 16)] = ((ti >> _LBITS) << _QBITS) | (ti & (_L // 4 - 1))

        def round_body(h, _):
            h0 = h * _HB
            cu = pltpu.async_copy(
                u4_hbm.at[utid.at[pl.ds(h0, _HB)]], ubuf, su)
            ci = pltpu.async_copy(
                i4_hbm.at[itid.at[pl.ds(h0, _HB)]], ibuf, si)
            cu.wait()
            ci.wait()
            pltpu.sync_copy(ubuf, ru_out.at[pl.ds(base + h0, _HB)])
            pltpu.sync_copy(ibuf, ri_out.at[pl.ds(base + h0, _HB)])
            return _

        lax.fori_loop(0, _RND, round_body, 0)

    return k(u4, i4, users, items)


def _mlp_body(ru_ref, ri_ref, uk_ref, ik_ref, w1e, b1r, w2, b2r, w3r, b3r,
              out_ref):
    ru = ru_ref[...]
    ri = ri_ref[...]
    uk = (uk_ref[...] >> _QBITS) & 3
    ik = (ik_ref[...] >> _QBITS) & 3
    lane = jax.lax.broadcasted_iota(jnp.int32, (1, 128), 1)
    parts = []
    for k in range(4):
        keep = (uk[:, None] == k) & (lane >> 5 == k)
        parts.append(jnp.where(keep, ru, 0.0))
    for k in range(4):
        keep = (ik[:, None] == k) & (lane >> 5 == k)
        parts.append(jnp.where(keep, ri, 0.0))
    xcat = jnp.concatenate(parts, axis=1)
    h1 = jnp.maximum(
        jnp.dot(xcat, w1e[...], preferred_element_type=jnp.float32)
        + b1r[...], 0.0)
    h2 = jnp.maximum(
        jnp.dot(h1, w2[...], preferred_element_type=jnp.float32) + b2r[...],
        0.0)
    out_ref[...] = jnp.sum(h2 * w3r[...], axis=1) + b3r[0]


def _mlp_tc(ru, ri, users, items, W1, b1, W2, b2, W3, b3):
    w1e = jnp.zeros((1024, 128), jnp.float32)
    for k in range(4):
        w1e = w1e.at[128 * k + 32 * k: 128 * k + 32 * k + 32].set(W1[:_EMB])
        w1e = w1e.at[512 + 128 * k + 32 * k: 512 + 128 * k + 32 * k + 32].set(
            W1[_EMB:])
    w3row = W3[:, 0]
    grid = (_BATCH // _BB,)
    return pl.pallas_call(
        _mlp_body,
        grid=grid,
        in_specs=[
            pl.BlockSpec((_BB, 128), lambda i: (i, 0)),
            pl.BlockSpec((_BB, 128), lambda i: (i, 0)),
            pl.BlockSpec((_BB,), lambda i: (i,)),
            pl.BlockSpec((_BB,), lambda i: (i,)),
            pl.BlockSpec((1024, 128), lambda i: (0, 0)),
            pl.BlockSpec((128,), lambda i: (0,)),
            pl.BlockSpec((128, 64), lambda i: (0, 0)),
            pl.BlockSpec((64,), lambda i: (0,)),
            pl.BlockSpec((64,), lambda i: (0,)),
            pl.BlockSpec((1,), lambda i: (0,)),
        ],
        out_specs=pl.BlockSpec((_BB,), lambda i: (i,)),
        out_shape=jax.ShapeDtypeStruct((_BATCH,), jnp.float32),
    )(ru, ri, users, items, w1e, b1, W2, b2, w3row, b3)


def kernel(users, items, user_emb, item_emb, W1, b1, W2, b2, W3, b3):
    users = users.astype(jnp.int32)
    items = items.astype(jnp.int32)
    u4 = _transpose_tc(user_emb.T)
    i4 = _transpose_tc(item_emb.T)
    ru, ri = _gather_sc(u4, i4, users, items)
    return _mlp_tc(ru, ri, users, items, W1, b1, W2, b2, W3, b3)


# speedup vs baseline: 3.5800x; 1.0130x over previous
import functools

import jax
import jax.numpy as jnp
from jax import lax
from jax.experimental import pallas as pl
from jax.experimental.pallas import tpu as pltpu
from jax.experimental.pallas import tpu_sc as plsc

_BATCH = 16384
_EMB = 32
_L = 65536
_LBITS = 16
_QBITS = _LBITS - 2
_NW = 32
_BPW = _BATCH // _NW
_RND = 2
_HB = _BPW // _RND
_BB = 1024


def _t_body(in_ref, out_ref):
    x = in_ref[...]
    L = x.shape[1]
    q = L // 4
    xs = jnp.concatenate([x[:, j * q:(j + 1) * q] for j in range(4)], axis=0)
    out_ref[...] = jnp.transpose(xs, (1, 0))


def _transpose_tc(embT, L=_L):
    n = embT.shape[1]
    nblk = (n + L - 1) // L
    grid = (nblk,)
    return pl.pallas_call(
        _t_body,
        grid=grid,
        in_specs=[pl.BlockSpec((_EMB, L), lambda m: (0, m))],
        out_specs=pl.BlockSpec((L // 4, 128), lambda m: (m, 0)),
        out_shape=jax.ShapeDtypeStruct((nblk * L // 4, 128), jnp.float32),
    )(embT)


def _gather_sc(u4, i4, users, items):
    mesh = plsc.VectorSubcoreMesh(core_axis_name="c", subcore_axis_name="s")

    @functools.partial(
        pl.kernel,
        mesh=mesh,
        out_type=[
            jax.ShapeDtypeStruct((_BATCH, 128), jnp.float32),
            jax.ShapeDtypeStruct((_BATCH, 128), jnp.float32),
        ],
        scratch_types=[
            pltpu.VMEM((_BPW,), jnp.int32),
            pltpu.VMEM((_BPW,), jnp.int32),
            pltpu.VMEM((_HB, 128), jnp.float32),
            pltpu.VMEM((_HB, 128), jnp.float32),
            pltpu.SemaphoreType.DMA,
            pltpu.SemaphoreType.DMA,
        ],
    )
    def k(u4_hbm, i4_hbm, users_hbm, items_hbm, ru_out, ri_out,
          utid, itid, ubuf, ibuf, su, si):
        wid = lax.axis_index("s") * 2 + lax.axis_index("c")
        base = wid * _BPW
        pltpu.sync_copy(users_hbm.at[pl.ds(base, _BPW)], utid)
        pltpu.sync_copy(items_hbm.at[pl.ds(base, _BPW)], itid)
        for g in range(_BPW // 16):
            tu = utid[pl.ds(g * 16, 16)]
            utid[pl.ds(g * 16, 16)] = ((tu >> _LBITS) << _QBITS) | (tu & (_L // 4 - 1))
            ti = itid[pl.ds(g * 16, 16)]
            itid[pl.ds(g * 16, 16)] = ((ti >> _LBITS) << _QBITS) | (ti & (_L // 4 - 1))

        def round_body(h, _):
            h0 = h * _HB
            cu = pltpu.async_copy(
                u4_hbm.at[utid.at[pl.ds(h0, _HB)]], ubuf, su)
            ci = pltpu.async_copy(
                i4_hbm.at[itid.at[pl.ds(h0, _HB)]], ibuf, si)
            cu.wait()
            ci.wait()
            pltpu.sync_copy(ubuf, ru_out.at[pl.ds(base + h0, _HB)])
            pltpu.sync_copy(ibuf, ri_out.at[pl.ds(base + h0, _HB)])
            return _

        lax.fori_loop(0, _RND, round_body, 0)

    return k(u4, i4, users, items)


def _mlp_body(ru_ref, ri_ref, uk_ref, ik_ref, w1e, b1r, w2, b2r, w3r, b3r,
              out_ref):
    ru = ru_ref[...]
    ri = ri_ref[...]
    uk = (uk_ref[...] >> _QBITS) & 3
    ik = (ik_ref[...] >> _QBITS) & 3
    lane = jax.lax.broadcasted_iota(jnp.int32, (1, 128), 1)
    parts = []
    for k in range(4):
        keep = (uk[:, None] == k) & (lane >> 5 == k)
        parts.append(jnp.where(keep, ru, 0.0))
    for k in range(4):
        keep = (ik[:, None] == k) & (lane >> 5 == k)
        parts.append(jnp.where(keep, ri, 0.0))
    xcat = jnp.concatenate(parts, axis=1)
    h1 = jnp.maximum(
        jnp.dot(xcat, w1e[...], preferred_element_type=jnp.float32)
        + b1r[...], 0.0)
    h2 = jnp.maximum(
        jnp.dot(h1, w2[...], preferred_element_type=jnp.float32) + b2r[...],
        0.0)
    out_ref[...] = jnp.sum(h2 * w3r[...], axis=1) + b3r[0]


def _mlp_tc(ru, ri, users, items, W1, b1, W2, b2, W3, b3):
    w1e = jnp.zeros((1024, 128), jnp.float32)
    for k in range(4):
        w1e = w1e.at[128 * k + 32 * k: 128 * k + 32 * k + 32].set(W1[:_EMB])
        w1e = w1e.at[512 + 128 * k + 32 * k: 512 + 128 * k + 32 * k + 32].set(
            W1[_EMB:])
    w3row = W3[:, 0]
    grid = (_BATCH // _BB,)
    return pl.pallas_call(
        _mlp_body,
        grid=grid,
        in_specs=[
            pl.BlockSpec((_BB, 128), lambda i: (i, 0)),
            pl.BlockSpec((_BB, 128), lambda i: (i, 0)),
            pl.BlockSpec((_BB,), lambda i: (i,)),
            pl.BlockSpec((_BB,), lambda i: (i,)),
            pl.BlockSpec((1024, 128), lambda i: (0, 0)),
            pl.BlockSpec((128,), lambda i: (0,)),
            pl.BlockSpec((128, 64), lambda i: (0, 0)),
            pl.BlockSpec((64,), lambda i: (0,)),
            pl.BlockSpec((64,), lambda i: (0,)),
            pl.BlockSpec((1,), lambda i: (0,)),
        ],
        out_specs=pl.BlockSpec((_BB,), lambda i: (i,)),
        out_shape=jax.ShapeDtypeStruct((_BATCH,), jnp.float32),
    )(ru, ri, users, items, w1e, b1, W2, b2, w3row, b3)


def kernel(users, items, user_emb, item_emb, W1, b1, W2, b2, W3, b3):
    users = users.astype(jnp.int32)
    items = items.astype(jnp.int32)
    u4 = _transpose_tc(user_emb.T)
    i4 = _transpose_tc(item_emb.T)
    ru, ri = _gather_sc(u4, i4, users, items)
    return _mlp_tc(ru, ri, users, items, W1, b1, W2, b2, W3, b3)


# speedup vs baseline: 3.7365x; 1.0437x over previous
import functools

import jax
import jax.numpy as jnp
from jax import lax
from jax.experimental import pallas as pl
from jax.experimental.pallas import tpu as pltpu
from jax.experimental.pallas import tpu_sc as plsc

_BATCH = 16384
_EMB = 32
_L = 65536
_LBITS = 16
_QBITS = _LBITS - 2
_NW = 32
_BPW = _BATCH // _NW
_RND = 2
_HB = _BPW // _RND
_BB = 1024


def _t_body(in_ref, out_ref):
    x = in_ref[...]
    L = x.shape[1]
    q = L // 4
    xs = jnp.concatenate([x[:, j * q:(j + 1) * q] for j in range(4)], axis=0)
    out_ref[...] = jnp.transpose(xs, (1, 0))


def _transpose_tc(embT, L=_L):
    n = embT.shape[1]
    nblk = (n + L - 1) // L
    grid = (nblk,)
    return pl.pallas_call(
        _t_body,
        grid=grid,
        in_specs=[pl.BlockSpec((_EMB, L), lambda m: (0, m))],
        out_specs=pl.BlockSpec((L // 4, 128), lambda m: (m, 0)),
        out_shape=jax.ShapeDtypeStruct((nblk * L // 4, 128), jnp.float32),
    )(embT)


def _gather_sc(t4, idx):
    mesh = plsc.VectorSubcoreMesh(core_axis_name="c", subcore_axis_name="s")

    @functools.partial(
        pl.kernel,
        mesh=mesh,
        out_type=jax.ShapeDtypeStruct((_BATCH, 128), jnp.float32),
        scratch_types=[
            pltpu.VMEM((_BPW,), jnp.int32),
            pltpu.VMEM((_HB, 128), jnp.float32),
            pltpu.VMEM((_HB, 128), jnp.float32),
            pltpu.SemaphoreType.DMA,
            pltpu.SemaphoreType.DMA,
        ],
    )
    def k(t4_hbm, idx_hbm, r_out, tid, buf0, buf1, s0, s1):
        wid = lax.axis_index("s") * 2 + lax.axis_index("c")
        base = wid * _BPW
        pltpu.sync_copy(idx_hbm.at[pl.ds(base, _BPW)], tid)
        for g in range(_BPW // 16):
            tv = tid[pl.ds(g * 16, 16)]
            tid[pl.ds(g * 16, 16)] = (
                ((tv >> _LBITS) << _QBITS) | (tv & (_L // 4 - 1)))
        c0 = pltpu.async_copy(t4_hbm.at[tid.at[pl.ds(0, _HB)]], buf0, s0)
        c1 = pltpu.async_copy(t4_hbm.at[tid.at[pl.ds(_HB, _HB)]], buf1, s1)
        c0.wait()
        pltpu.sync_copy(buf0, r_out.at[pl.ds(base, _HB)])
        c1.wait()
        pltpu.sync_copy(buf1, r_out.at[pl.ds(base + _HB, _HB)])

    return k(t4, idx)


def _mlp_body(ru_ref, ri_ref, uk_ref, ik_ref, w1e, b1r, w2, b2r, w3r, b3r,
              out_ref):
    ru = ru_ref[...]
    ri = ri_ref[...]
    uk = (uk_ref[...] >> _QBITS) & 3
    ik = (ik_ref[...] >> _QBITS) & 3
    lane = jax.lax.broadcasted_iota(jnp.int32, (1, 128), 1) >> 5
    xu = jnp.where(lane == uk[:, None], ru, 0.0)
    xi = jnp.where(lane == ik[:, None], ri, 0.0)
    xcat = jnp.concatenate([xu, xi], axis=1)
    h1 = jnp.maximum(
        jnp.dot(xcat, w1e[...], preferred_element_type=jnp.float32)
        + b1r[...], 0.0)
    h2 = jnp.maximum(
        jnp.dot(h1, w2[...], preferred_element_type=jnp.float32) + b2r[...],
        0.0)
    out_ref[...] = jnp.sum(h2 * w3r[...], axis=1) + b3r[0]


def _mlp_tc(ru, ri, users, items, W1, b1, W2, b2, W3, b3):
    w1e = jnp.concatenate([W1[:_EMB]] * 4 + [W1[_EMB:]] * 4, axis=0)
    w3row = W3[:, 0]
    grid = (_BATCH // _BB,)
    return pl.pallas_call(
        _mlp_body,
        grid=grid,
        in_specs=[
            pl.BlockSpec((_BB, 128), lambda i: (i, 0)),
            pl.BlockSpec((_BB, 128), lambda i: (i, 0)),
            pl.BlockSpec((_BB,), lambda i: (i,)),
            pl.BlockSpec((_BB,), lambda i: (i,)),
            pl.BlockSpec((256, 128), lambda i: (0, 0)),
            pl.BlockSpec((128,), lambda i: (0,)),
            pl.BlockSpec((128, 64), lambda i: (0, 0)),
            pl.BlockSpec((64,), lambda i: (0,)),
            pl.BlockSpec((64,), lambda i: (0,)),
            pl.BlockSpec((1,), lambda i: (0,)),
        ],
        out_specs=pl.BlockSpec((_BB,), lambda i: (i,)),
        out_shape=jax.ShapeDtypeStruct((_BATCH,), jnp.float32),
    )(ru, ri, users, items, w1e, b1, W2, b2, w3row, b3)


def kernel(users, items, user_emb, item_emb, W1, b1, W2, b2, W3, b3):
    users = users.astype(jnp.int32)
    items = items.astype(jnp.int32)
    i4 = _transpose_tc(item_emb.T)
    ri = _gather_sc(i4, items)
    u4 = _transpose_tc(user_emb.T)
    ru = _gather_sc(u4, users)
    return _mlp_tc(ru, ri, users, items, W1, b1, W2, b2, W3, b3)


# speedup vs baseline: 4.1257x; 1.1041x over previous
import functools

import jax
import jax.numpy as jnp
from jax import lax
from jax.experimental import pallas as pl
from jax.experimental.pallas import tpu as pltpu
from jax.experimental.pallas import tpu_sc as plsc

_BATCH = 16384
_EMB = 32
_L = 65536
_LBITS = 16
_QBITS = _LBITS - 2
_NW = 32
_BPW = _BATCH // _NW
_RND = 2
_HB = _BPW // _RND
_BB = 1024


def _t_body(in_ref, out_ref):
    x = in_ref[...]
    L = x.shape[1]
    q = L // 4
    xs = jnp.concatenate([x[:, j * q:(j + 1) * q] for j in range(4)], axis=0)
    out_ref[...] = jnp.transpose(xs, (1, 0))


def _transpose_tc(embT, L=_L):
    n = embT.shape[1]
    nblk = (n + L - 1) // L
    grid = (nblk,)
    return pl.pallas_call(
        _t_body,
        grid=grid,
        in_specs=[pl.BlockSpec((_EMB, L), lambda m: (0, m))],
        out_specs=pl.BlockSpec((L // 4, 128), lambda m: (m, 0)),
        out_shape=jax.ShapeDtypeStruct((nblk * L // 4, 128), jnp.float32),
    )(embT)


def _gather_sc(t4, idx):
    mesh = plsc.VectorSubcoreMesh(core_axis_name="c", subcore_axis_name="s")

    @functools.partial(
        pl.kernel,
        mesh=mesh,
        out_type=jax.ShapeDtypeStruct((_BATCH, 128), jnp.float32),
        scratch_types=[
            pltpu.VMEM((_BPW,), jnp.int32),
            pltpu.VMEM((_HB, 128), jnp.float32),
            pltpu.VMEM((_HB, 128), jnp.float32),
            pltpu.SemaphoreType.DMA,
            pltpu.SemaphoreType.DMA,
        ],
    )
    def k(t4_hbm, idx_hbm, r_out, tid, buf0, buf1, s0, s1):
        wid = lax.axis_index("s") * 2 + lax.axis_index("c")
        base = wid * _BPW
        pltpu.sync_copy(idx_hbm.at[pl.ds(base, _BPW)], tid)
        for g in range(_BPW // 16):
            tv = tid[pl.ds(g * 16, 16)]
            tid[pl.ds(g * 16, 16)] = (
                ((tv >> _LBITS) << _QBITS) | (tv & (_L // 4 - 1)))
        c0 = pltpu.async_copy(t4_hbm.at[tid.at[pl.ds(0, _HB)]], buf0, s0)
        c1 = pltpu.async_copy(t4_hbm.at[tid.at[pl.ds(_HB, _HB)]], buf1, s1)
        c0.wait()
        pltpu.sync_copy(buf0, r_out.at[pl.ds(base, _HB)])
        c1.wait()
        pltpu.sync_copy(buf1, r_out.at[pl.ds(base + _HB, _HB)])

    return k(t4, idx)


def _mlp_body(ru_ref, ri_ref, uk_ref, ik_ref, w1e, b1r, w2, b2r, w3r, b3r,
              out_ref):
    ru = ru_ref[...]
    ri = ri_ref[...]
    uk = (uk_ref[...] >> _QBITS) & 3
    ik = (ik_ref[...] >> _QBITS) & 3
    lane = jax.lax.broadcasted_iota(jnp.int32, (1, 128), 1) >> 5
    xu = jnp.where(lane == uk[:, None], ru, 0.0)
    xi = jnp.where(lane == ik[:, None], ri, 0.0)
    xcat = jnp.concatenate([xu, xi], axis=1)
    h1 = jnp.maximum(
        jnp.dot(xcat, w1e[...], preferred_element_type=jnp.float32)
        + b1r[...], 0.0)
    h2 = jnp.maximum(
        jnp.dot(h1, w2[...], preferred_element_type=jnp.float32) + b2r[...],
        0.0)
    res = lax.dot_general(
        w3r[...].reshape(1, 64), h2, (((1,), (1,)), ((), ())),
        preferred_element_type=jnp.float32)
    out_ref[...] = res.reshape(res.shape[1]) + b3r[0]


def _mlp_tc(ru, ri, users, items, W1, b1, W2, b2, W3, b3):
    w1e = jnp.concatenate([W1[:_EMB]] * 4 + [W1[_EMB:]] * 4, axis=0)
    w3row = W3[:, 0]
    grid = (_BATCH // _BB,)
    return pl.pallas_call(
        _mlp_body,
        grid=grid,
        in_specs=[
            pl.BlockSpec((_BB, 128), lambda i: (i, 0)),
            pl.BlockSpec((_BB, 128), lambda i: (i, 0)),
            pl.BlockSpec((_BB,), lambda i: (i,)),
            pl.BlockSpec((_BB,), lambda i: (i,)),
            pl.BlockSpec((256, 128), lambda i: (0, 0)),
            pl.BlockSpec((128,), lambda i: (0,)),
            pl.BlockSpec((128, 64), lambda i: (0, 0)),
            pl.BlockSpec((64,), lambda i: (0,)),
            pl.BlockSpec((64,), lambda i: (0,)),
            pl.BlockSpec((1,), lambda i: (0,)),
        ],
        out_specs=pl.BlockSpec((_BB,), lambda i: (i,)),
        out_shape=jax.ShapeDtypeStruct((_BATCH,), jnp.float32),
    )(ru, ri, users, items, w1e, b1, W2, b2, w3row, b3)


def kernel(users, items, user_emb, item_emb, W1, b1, W2, b2, W3, b3):
    users = users.astype(jnp.int32)
    items = items.astype(jnp.int32)
    i4 = _transpose_tc(item_emb.T)
    ri = _gather_sc(i4, items)
    u4 = _transpose_tc(user_emb.T)
    ru = _gather_sc(u4, users)
    return _mlp_tc(ru, ri, users, items, W1, b1, W2, b2, W3, b3)


# speedup vs baseline: 4.2720x; 1.0355x over previous
import functools

import jax
import jax.numpy as jnp
from jax import lax
from jax.experimental import pallas as pl
from jax.experimental.pallas import tpu as pltpu
from jax.experimental.pallas import tpu_sc as plsc

_BATCH = 16384
_EMB = 32
_L = 65536
_LBITS = 16
_QBITS = _LBITS - 2
_NW = 32
_BPW = _BATCH // _NW
_RND = 2
_HB = _BPW // _RND
_BB = 2048


def _t_body(in_ref, out_ref):
    x = in_ref[...]
    L = x.shape[1]
    q = L // 4
    xs = jnp.concatenate([x[:, j * q:(j + 1) * q] for j in range(4)], axis=0)
    out_ref[...] = jnp.transpose(xs, (1, 0))


def _transpose_tc(embT, L=_L):
    n = embT.shape[1]
    nblk = (n + L - 1) // L
    grid = (nblk,)
    return pl.pallas_call(
        _t_body,
        grid=grid,
        in_specs=[pl.BlockSpec((_EMB, L), lambda m: (0, m))],
        out_specs=pl.BlockSpec((L // 4, 128), lambda m: (m, 0)),
        out_shape=jax.ShapeDtypeStruct((nblk * L // 4, 128), jnp.float32),
    )(embT)


def _gather_sc(t4, idx):
    mesh = plsc.VectorSubcoreMesh(core_axis_name="c", subcore_axis_name="s")

    @functools.partial(
        pl.kernel,
        mesh=mesh,
        out_type=jax.ShapeDtypeStruct((_BATCH, 128), jnp.float32),
        scratch_types=[
            pltpu.VMEM((_BPW,), jnp.int32),
            pltpu.VMEM((_HB, 128), jnp.float32),
            pltpu.VMEM((_HB, 128), jnp.float32),
            pltpu.SemaphoreType.DMA,
            pltpu.SemaphoreType.DMA,
        ],
    )
    def k(t4_hbm, idx_hbm, r_out, tid, buf0, buf1, s0, s1):
        wid = lax.axis_index("s") * 2 + lax.axis_index("c")
        base = wid * _BPW
        pltpu.sync_copy(idx_hbm.at[pl.ds(base, _BPW)], tid)
        for g in range(_BPW // 16):
            tv = tid[pl.ds(g * 16, 16)]
            tid[pl.ds(g * 16, 16)] = (
                ((tv >> _LBITS) << _QBITS) | (tv & (_L // 4 - 1)))
        c0 = pltpu.async_copy(t4_hbm.at[tid.at[pl.ds(0, _HB)]], buf0, s0)
        c1 = pltpu.async_copy(t4_hbm.at[tid.at[pl.ds(_HB, _HB)]], buf1, s1)
        c0.wait()
        pltpu.sync_copy(buf0, r_out.at[pl.ds(base, _HB)])
        c1.wait()
        pltpu.sync_copy(buf1, r_out.at[pl.ds(base + _HB, _HB)])

    return k(t4, idx)


def _mlp_body(ru_ref, ri_ref, uk_ref, ik_ref, w1e, b1r, w2, b2r, w3r, b3r,
              out_ref):
    ru = ru_ref[...]
    ri = ri_ref[...]
    uk = (uk_ref[...] >> _QBITS) & 3
    ik = (ik_ref[...] >> _QBITS) & 3
    lane = jax.lax.broadcasted_iota(jnp.int32, (1, 128), 1) >> 5
    xu = jnp.where(lane == uk[:, None], ru, 0.0)
    xi = jnp.where(lane == ik[:, None], ri, 0.0)
    xcat = jnp.concatenate([xu, xi], axis=1)
    h1 = jnp.maximum(
        jnp.dot(xcat, w1e[...], preferred_element_type=jnp.float32)
        + b1r[...], 0.0)
    h2 = jnp.maximum(
        jnp.dot(h1, w2[...], preferred_element_type=jnp.float32) + b2r[...],
        0.0)
    res = lax.dot_general(
        w3r[...].reshape(1, 64), h2, (((1,), (1,)), ((), ())),
        preferred_element_type=jnp.float32)
    out_ref[...] = res.reshape(res.shape[1]) + b3r[0]


def _mlp_tc(ru, ri, users, items, W1, b1, W2, b2, W3, b3):
    w1e = jnp.concatenate([W1[:_EMB]] * 4 + [W1[_EMB:]] * 4, axis=0)
    w3row = W3[:, 0]
    grid = (_BATCH // _BB,)
    return pl.pallas_call(
        _mlp_body,
        grid=grid,
        in_specs=[
            pl.BlockSpec((_BB, 128), lambda i: (i, 0)),
            pl.BlockSpec((_BB, 128), lambda i: (i, 0)),
            pl.BlockSpec((_BB,), lambda i: (i,)),
            pl.BlockSpec((_BB,), lambda i: (i,)),
            pl.BlockSpec((256, 128), lambda i: (0, 0)),
            pl.BlockSpec((128,), lambda i: (0,)),
            pl.BlockSpec((128, 64), lambda i: (0, 0)),
            pl.BlockSpec((64,), lambda i: (0,)),
            pl.BlockSpec((64,), lambda i: (0,)),
            pl.BlockSpec((1,), lambda i: (0,)),
        ],
        out_specs=pl.BlockSpec((_BB,), lambda i: (i,)),
        out_shape=jax.ShapeDtypeStruct((_BATCH,), jnp.float32),
    )(ru, ri, users, items, w1e, b1, W2, b2, w3row, b3)


def kernel(users, items, user_emb, item_emb, W1, b1, W2, b2, W3, b3):
    users = users.astype(jnp.int32)
    items = items.astype(jnp.int32)
    i4 = _transpose_tc(item_emb.T)
    ri = _gather_sc(i4, items)
    u4 = _transpose_tc(user_emb.T)
    ru = _gather_sc(u4, users)
    return _mlp_tc(ru, ri, users, items, W1, b1, W2, b2, W3, b3)
